# Initial kernel scaffold; baseline (speedup 1.0000x reference)
import jax
import jax.numpy as jnp
from jax.experimental import pallas as pl


def kernel(x, table):
    raise NotImplementedError("write your pallas kernel here")


# speedup vs baseline: 1.4368x; 1.4368x over previous
import functools

import jax
import jax.numpy as jnp
from jax import lax
from jax.experimental import pallas as pl
from jax.experimental.pallas import tpu as pltpu
from jax.experimental.pallas import tpu_sc as plsc

D = 32
NW = 32
CHUNK = 128
N_CHUNKS = 104
B_PER_W = CHUNK * N_CHUNKS

_mesh = plsc.VectorSubcoreMesh(core_axis_name="c", subcore_axis_name="s")


@functools.partial(
    pl.kernel,
    out_type=jax.ShapeDtypeStruct((NW * B_PER_W, D), jnp.float32),
    mesh=_mesh,
    scratch_types=[
        pltpu.VMEM((N_CHUNKS, CHUNK), jnp.int32),
        pltpu.VMEM((CHUNK, D), jnp.float32),
        pltpu.SemaphoreType.DMA,
    ],
    compiler_params=pltpu.CompilerParams(use_tc_tiling_on_sc=False),
)
def _gather(x_hbm, table_hbm, out_hbm, idx_v, rows_v, sem):
    wid = lax.axis_index("s") * 2 + lax.axis_index("c")
    base = pl.multiple_of(wid * B_PER_W, CHUNK)
    pltpu.sync_copy(x_hbm.at[wid], idx_v)

    def body(j, carry):
        pltpu.async_copy(table_hbm.at[idx_v.at[j]], rows_v, sem).wait()
        pltpu.sync_copy(rows_v, out_hbm.at[pl.ds(base + j * CHUNK, CHUNK)])
        return carry

    lax.fori_loop(0, N_CHUNKS, body, 0)


def kernel(x, table):
    b0, b1 = x.shape
    xr = x.reshape(NW, N_CHUNKS, CHUNK)
    out = _gather(xr, table)
    return out.reshape(b0, b1, D)


# speedup vs baseline: 1.5765x; 1.0973x over previous
import jax
import jax.numpy as jnp
from jax import lax
from jax.experimental import pallas as pl
from jax.experimental.pallas import tpu as pltpu
from jax.experimental.pallas import tpu_sc as plsc

D = 32
NW = 32
CHUNK = 128
N_CHUNKS = 104
B_PER_W = CHUNK * N_CHUNKS
K = 4
GR = K * CHUNK
G = N_CHUNKS // K

_mesh = plsc.VectorSubcoreMesh(core_axis_name="c", subcore_axis_name="s")


def _gather_body(x_hbm, table_hbm, out_hbm, idx_v, rows_v, gsemA, gsemB,
                 osemA, osemB):
    wid = lax.axis_index("s") * 2 + lax.axis_index("c")
    base = pl.multiple_of(wid * B_PER_W, CHUNK)
    pltpu.sync_copy(x_hbm.at[wid], idx_v)
    gsem = (gsemA, gsemB)
    osem = (osemA, osemB)

    def fire(g, buf):
        for k in range(K):
            pltpu.async_copy(
                table_hbm.at[idx_v.at[g * K + k]],
                rows_v.at[buf, pl.ds(k * CHUNK, CHUNK)],
                gsem[buf])

    def drain_gathers(buf):
        pltpu.make_async_copy(
            out_hbm.at[pl.ds(base, GR)], rows_v.at[buf], gsem[buf]).wait()

    def outcopy(g, buf):
        pltpu.async_copy(
            rows_v.at[buf], out_hbm.at[pl.ds(base + g * GR, GR)], osem[buf])

    def wait_outcopy(g, buf):
        pltpu.make_async_copy(
            rows_v.at[buf], out_hbm.at[pl.ds(base + g * GR, GR)],
            osem[buf]).wait()

    fire(0, 0)
    fire(1, 1)
    drain_gathers(0)
    outcopy(0, 0)

    def body(g, buf):
        wait_outcopy(g - 2, buf)
        fire(g, buf)
        drain_gathers(buf ^ 1)
        outcopy(g - 1, buf ^ 1)

    def loop_body(i, carry):
        g0 = 2 + 2 * i
        body(g0, 0)
        body(g0 + 1, 1)
        return carry

    lax.fori_loop(0, (G - 2) // 2, loop_body, 0)

    drain_gathers(1)
    outcopy(G - 1, 1)
    wait_outcopy(G - 2, 0)
    wait_outcopy(G - 1, 1)


_gather = pl.kernel(
    _gather_body,
    out_type=jax.ShapeDtypeStruct((NW * B_PER_W, D), jnp.float32),
    mesh=_mesh,
    scratch_types=[
        pltpu.VMEM((N_CHUNKS, CHUNK), jnp.int32),
        pltpu.VMEM((2, GR, D), jnp.float32),
        pltpu.SemaphoreType.DMA,
        pltpu.SemaphoreType.DMA,
        pltpu.SemaphoreType.DMA,
        pltpu.SemaphoreType.DMA,
    ],
    compiler_params=pltpu.CompilerParams(use_tc_tiling_on_sc=False),
)


def kernel(x, table):
    b0, b1 = x.shape
    xr = x.reshape(NW, N_CHUNKS, CHUNK)
    out = _gather(xr, table)
    return out.reshape(b0, b1, D)
